# baseline (speedup 1.0000x reference)
import jax
import jax.numpy as jnp
from jax.experimental import pallas as pl

_B, _V, _F = 8, 2048, 64
_K = 40
_D = 4
_NF = 64
_P = 32


def _final_body(x_ref, c_ref, w_ref, b_ref, o_ref):
    u = jnp.concatenate([x_ref[...], c_ref[...]], axis=-1)
    o_ref[...] = jnp.tanh(
        jnp.dot(u, w_ref[...], preferred_element_type=jnp.float32) + b_ref[...]
    )


def kernel(x, W_flr, b_flr, W_s, b_s, W_out, b_out):
    coordinates = jnp.dot(x, W_s) + b_s
    dotA = jnp.sum(coordinates * coordinates, axis=-1)[:, :, None]
    dotB = jnp.sum(coordinates * coordinates, axis=-1)[:, None, :]
    cross = jnp.einsum('bid,bjd->bij', coordinates, coordinates)
    dmat = jnp.maximum(dotA + dotB - 2.0 * cross, 0.0)
    ranked_distances, ranked_indices = jax.lax.top_k(-dmat, _K)
    neighbour_indices = ranked_indices[:, :, 1:]
    distance = -ranked_distances[:, :, 1:]
    weights = jnp.exp(-jnp.abs(distance * 10.0))[..., None]
    features = jnp.dot(x, W_flr) + b_flr
    neighbour_features = jax.vmap(lambda f, i: f[i])(features, neighbour_indices)
    neighbour_features = neighbour_features * weights
    neighbours_max = jnp.max(neighbour_features, axis=2)
    neighbours_mean = jnp.mean(neighbour_features, axis=2)
    collected = jnp.concatenate([neighbours_max, neighbours_mean], axis=-1)

    xf = x.reshape(_B * _V, _F)
    cf = collected.reshape(_B * _V, 2 * _P)
    out = pl.pallas_call(
        _final_body,
        out_shape=jax.ShapeDtypeStruct((_B * _V, _NF), jnp.float32),
        grid=(_B,),
        in_specs=[
            pl.BlockSpec((_V, _F), lambda i: (i, 0)),
            pl.BlockSpec((_V, 2 * _P), lambda i: (i, 0)),
            pl.BlockSpec((_F + 2 * _P, _NF), lambda i: (0, 0)),
            pl.BlockSpec((1, _NF), lambda i: (0, 0)),
        ],
        out_specs=pl.BlockSpec((_V, _NF), lambda i: (i, 0)),
    )(xf, cf, W_out, b_out.reshape(1, _NF))
    return out.reshape(_B, _V, _NF)


# speedup vs baseline: 1.3754x; 1.3754x over previous
import functools

import jax
import jax.numpy as jnp
from jax.experimental import pallas as pl
from jax.experimental.pallas import tpu as pltpu

_B, _V, _F = 8, 2048, 64
_K = 40
_KN = _K - 1
_D = 4
_NF = 64
_P = 32
_R = 512
_T = _V // _R
_KPAD = 64

_IDX_BITS = 11
_IDX_MASK = (1 << _IDX_BITS) - 1
_KEY_MAX = 0x7FFFFFFF


def _topk_body(x_ref, xt_ref, ws_ref, bs_ref, wf_ref, bf_ref,
               feats_ref, nbr_ref, wgt_ref):
    t = pl.program_id(1)
    x_all = x_ref[0]

    @pl.when(t == 0)
    def _():
        feats_ref[0] = jnp.dot(x_all, wf_ref[...],
                               preferred_element_type=jnp.float32) + bf_ref[...]

    coords = jnp.dot(x_all, ws_ref[...],
                     preferred_element_type=jnp.float32) + bs_ref[...]
    ct = jnp.dot(xt_ref[0], ws_ref[...],
                 preferred_element_type=jnp.float32) + bs_ref[...]
    cT = coords.T
    da = jnp.sum(ct * ct, axis=1, keepdims=True)
    db = jnp.sum(cT * cT, axis=0, keepdims=True)
    def _round_bf16(v):
        b = jax.lax.bitcast_convert_type(v, jnp.int32)
        r = (b + 0x7FFF + jnp.bitwise_and(jax.lax.shift_right_logical(b, 16), 1))
        r = jnp.bitwise_and(r, jnp.int32(-65536))
        return jax.lax.bitcast_convert_type(r, jnp.float32)

    cbT = _round_bf16(cT)
    ctb = _round_bf16(ct)
    cross = ctb[:, 0:1] * cbT[0:1, :]
    for d in range(1, _D):
        cross = cross + ctb[:, d:d + 1] * cbT[d:d + 1, :]
    dist = jnp.maximum(da + db - 2.0 * cross, 0.0)

    cols = jax.lax.broadcasted_iota(jnp.int32, (_R, _V), 1)
    bits = jax.lax.bitcast_convert_type(dist, jnp.int32)
    keys = jnp.bitwise_or(jnp.bitwise_and(bits, ~jnp.int32(_IDX_MASK)), cols)

    m_prev = jnp.full((_R, 1), -1, jnp.int32)
    for k in range(_K):
        cand = jnp.where(keys > m_prev, keys, _KEY_MAX)
        m = jnp.min(cand, axis=1, keepdims=True)
        if k > 0:
            idx = jnp.bitwise_and(m, jnp.int32(_IDX_MASK))
            dval = jax.lax.bitcast_convert_type(
                jnp.bitwise_and(m, ~jnp.int32(_IDX_MASK)), jnp.float32)
            w = jnp.exp(-10.0 * dval)
            nbr_ref[0, k - 1, :] = idx[:, 0]
            wgt_ref[0, k - 1, :] = w[:, 0]
        m_prev = m


def _topk_stage(x, W_s, b_s, W_flr, b_flr):
    return pl.pallas_call(
        _topk_body,
        grid=(_B, _T),
        in_specs=[
            pl.BlockSpec((1, _V, _F), lambda b, t: (b, 0, 0)),
            pl.BlockSpec((1, _R, _F), lambda b, t: (b, t, 0)),
            pl.BlockSpec((_F, _D), lambda b, t: (0, 0)),
            pl.BlockSpec((1, _D), lambda b, t: (0, 0)),
            pl.BlockSpec((_F, _P), lambda b, t: (0, 0)),
            pl.BlockSpec((1, _P), lambda b, t: (0, 0)),
        ],
        out_specs=[
            pl.BlockSpec((1, _V, _P), lambda b, t: (b, 0, 0)),
            pl.BlockSpec((1, _KPAD, _R), lambda b, t: (b, 0, t)),
            pl.BlockSpec((1, _KPAD, _R), lambda b, t: (b, 0, t)),
        ],
        out_shape=[
            jax.ShapeDtypeStruct((_B, _V, _P), jnp.float32),
            jax.ShapeDtypeStruct((_B, _KPAD, _V), jnp.int32),
            jax.ShapeDtypeStruct((_B, _KPAD, _V), jnp.float32),
        ],
    )(x, x, W_s, b_s.reshape(1, _D), W_flr, b_flr.reshape(1, _P))


def _final_body(x_ref, c_ref, w_ref, b_ref, o_ref):
    u = jnp.concatenate([x_ref[...], c_ref[...]], axis=-1)
    o_ref[...] = jnp.tanh(
        jnp.dot(u, w_ref[...], preferred_element_type=jnp.float32) + b_ref[...]
    )


def _final_stage(x, collected, W_out, b_out):
    xf = x.reshape(_B * _V, _F)
    cf = collected.reshape(_B * _V, 2 * _P)
    out = pl.pallas_call(
        _final_body,
        out_shape=jax.ShapeDtypeStruct((_B * _V, _NF), jnp.float32),
        grid=(_B,),
        in_specs=[
            pl.BlockSpec((_V, _F), lambda i: (i, 0)),
            pl.BlockSpec((_V, 2 * _P), lambda i: (i, 0)),
            pl.BlockSpec((_F + 2 * _P, _NF), lambda i: (0, 0)),
            pl.BlockSpec((1, _NF), lambda i: (0, 0)),
        ],
        out_specs=pl.BlockSpec((_V, _NF), lambda i: (i, 0)),
    )(xf, cf, W_out, b_out.reshape(1, _NF))
    return out.reshape(_B, _V, _NF)


def kernel(x, W_flr, b_flr, W_s, b_s, W_out, b_out):
    feats, nbr, wgt = _topk_stage(x, W_s, b_s, W_flr, b_flr)
    idx = nbr[:, :_KN, :]
    w = wgt[:, :_KN, :]
    nf = jax.vmap(lambda f, i: f[i])(feats, idx)
    nf = nf * w[..., None]
    nmax = jnp.max(nf, axis=1)
    nmean = jnp.sum(nf, axis=1) * (1.0 / _KN)
    collected = jnp.concatenate([nmax, nmean], axis=-1)
    return _final_stage(x, collected, W_out, b_out)


# speedup vs baseline: 8.9722x; 6.5233x over previous
import dataclasses
import functools

import jax
import jax.numpy as jnp
from jax.experimental import pallas as pl
from jax.experimental.pallas import tpu as pltpu
from jax.experimental.pallas import tpu_sc as plsc

_B, _V, _F = 8, 2048, 64
_K = 40
_KN = _K - 1
_D = 4
_NF = 64
_P = 32
_R = 512
_T = _V // _R
_KPAD = 64

_IDX_BITS = 11
_IDX_MASK = (1 << _IDX_BITS) - 1
_KEY_MAX = 0x7FFFFFFF


def _topk_body(x_ref, xt_ref, ws_ref, bs_ref, wf_ref, bf_ref,
               feats_ref, nbr_ref, wgt_ref):
    t = pl.program_id(1)
    x_all = x_ref[0]

    f = jnp.dot(xt_ref[0], wf_ref[...],
                preferred_element_type=jnp.float32) + bf_ref[...]
    feats_ref[0, 0] = f[:, :_P // 2]
    feats_ref[0, 1] = f[:, _P // 2:]

    coords = jnp.dot(x_all, ws_ref[...],
                     preferred_element_type=jnp.float32) + bs_ref[...]
    ct = jnp.dot(xt_ref[0], ws_ref[...],
                 preferred_element_type=jnp.float32) + bs_ref[...]
    cT = coords.T
    da = jnp.sum(ct * ct, axis=1, keepdims=True)
    db = jnp.sum(cT * cT, axis=0, keepdims=True)
    def _round_bf16(v):
        b = jax.lax.bitcast_convert_type(v, jnp.int32)
        r = (b + 0x7FFF + jnp.bitwise_and(jax.lax.shift_right_logical(b, 16), 1))
        r = jnp.bitwise_and(r, jnp.int32(-65536))
        return jax.lax.bitcast_convert_type(r, jnp.float32)

    cbT = _round_bf16(cT)
    ctb = _round_bf16(ct)
    cross = ctb[:, 0:1] * cbT[0:1, :]
    for d in range(1, _D):
        cross = cross + ctb[:, d:d + 1] * cbT[d:d + 1, :]
    dist = jnp.maximum(da + db - 2.0 * cross, 0.0)

    cols = jax.lax.broadcasted_iota(jnp.int32, (_R, _V), 1)
    bits = jax.lax.bitcast_convert_type(dist, jnp.int32)
    keys = jnp.bitwise_or(jnp.bitwise_and(bits, ~jnp.int32(_IDX_MASK)), cols)

    m_prev = jnp.full((_R, 1), -1, jnp.int32)
    for k in range(_K):
        cand = jnp.where(keys > m_prev, keys, _KEY_MAX)
        m = jnp.min(cand, axis=1, keepdims=True)
        if k > 0:
            idx = jnp.bitwise_and(m, jnp.int32(_IDX_MASK))
            dval = jax.lax.bitcast_convert_type(
                jnp.bitwise_and(m, ~jnp.int32(_IDX_MASK)), jnp.float32)
            w = jnp.exp(-10.0 * dval)
            nbr_ref[0, k - 1, :] = idx[:, 0]
            wgt_ref[0, k - 1, :] = w[:, 0]
        m_prev = m


def _topk_stage(x, W_s, b_s, W_flr, b_flr):
    return pl.pallas_call(
        _topk_body,
        grid=(_B, _T),
        compiler_params=pltpu.CompilerParams(
            dimension_semantics=("parallel", "parallel")),
        in_specs=[
            pl.BlockSpec((1, _V, _F), lambda b, t: (b, 0, 0)),
            pl.BlockSpec((1, _R, _F), lambda b, t: (b, t, 0)),
            pl.BlockSpec((_F, _D), lambda b, t: (0, 0)),
            pl.BlockSpec((1, _D), lambda b, t: (0, 0)),
            pl.BlockSpec((_F, _P), lambda b, t: (0, 0)),
            pl.BlockSpec((1, _P), lambda b, t: (0, 0)),
        ],
        out_specs=[
            pl.BlockSpec((1, 2, _R, _P // 2), lambda b, t: (b, 0, t, 0)),
            pl.BlockSpec((1, _KPAD, _R), lambda b, t: (b, 0, t)),
            pl.BlockSpec((1, _KPAD, _R), lambda b, t: (b, 0, t)),
        ],
        out_shape=[
            jax.ShapeDtypeStruct((_B, 2, _V, _P // 2), jnp.float32),
            jax.ShapeDtypeStruct((_B, _KPAD, _V), jnp.int32),
            jax.ShapeDtypeStruct((_B, _KPAD, _V), jnp.float32),
        ],
    )(x, x, W_s, b_s.reshape(1, _D), W_flr, b_flr.reshape(1, _P))


_NC = 2
_NS = 16
_NW = _NC * _NS
_PH = _P // 2
_SEG = 1024
_G = 16


_CB = 128


_PB = 8


def _agg_sc_body(feats_hbm, nbr_hbm, wgt_hbm, out_hbm,
                 table_v, idx_v, wgt_v, out_v):
    c = jax.lax.axis_index("c")
    s = jax.lax.axis_index("s")
    wid = s * _NC + c
    b = wid // 4
    half = (wid // 2) % 2
    vbase = (wid % 2) * _SEG
    pltpu.sync_copy(feats_hbm.at[b, half], table_v)
    lane = jax.lax.broadcasted_iota(jnp.int32, (_G,), 0)

    @pl.loop(0, _SEG // _CB)
    def _(cchunk):
        cb = vbase + cchunk * _CB
        pltpu.sync_copy(nbr_hbm.at[b, pl.ds(0, _KPAD), pl.ds(cb, _CB)], idx_v)
        pltpu.sync_copy(wgt_hbm.at[b, pl.ds(0, _KPAD), pl.ds(cb, _CB)], wgt_v)

        @pl.loop(0, _CB // _G)
        def _(sub):
            col = sub * _G
            rows = lane + col
            for pb in range(_PH // _PB):
                def kbody(k, accs):
                    idx16 = idx_v[k, pl.ds(col, _G)]
                    w16 = wgt_v[k, pl.ds(col, _G)]
                    new = []
                    for j in range(_PB):
                        p = pb * _PB + j
                        g = plsc.load_gather(
                            table_v, [idx16, jnp.full((_G,), p, jnp.int32)])
                        wf = w16 * g
                        new.append(jnp.maximum(accs[2 * j], wf))
                        new.append(accs[2 * j + 1] + wf)
                    return tuple(new)

                init = []
                for j in range(_PB):
                    init.append(jnp.full((_G,), -jnp.inf, jnp.float32))
                    init.append(jnp.zeros((_G,), jnp.float32))
                accs = jax.lax.fori_loop(0, _KN, kbody, tuple(init))
                for j in range(_PB):
                    p = pb * _PB + j
                    plsc.store_scatter(
                        out_v, [rows, jnp.full((_G,), p, jnp.int32)],
                        accs[2 * j])
                    plsc.store_scatter(
                        out_v, [rows, jnp.full((_G,), _PH + p, jnp.int32)],
                        accs[2 * j + 1] * (1.0 / _KN))

        pltpu.sync_copy(out_v, out_hbm.at[half, pl.ds(b * _V + cb, _CB)])


def _agg_stage(feats, nbr, wgt):
    mesh = plsc.VectorSubcoreMesh(core_axis_name="c", subcore_axis_name="s")
    cp = pltpu.CompilerParams()
    if "needs_layout_passes" in pltpu.CompilerParams.__dataclass_fields__:
        cp = dataclasses.replace(cp, needs_layout_passes=False)
    if "use_tc_tiling_on_sc" in pltpu.CompilerParams.__dataclass_fields__:
        cp = dataclasses.replace(cp, use_tc_tiling_on_sc=False)
    f = pl.kernel(
        _agg_sc_body,
        out_type=jax.ShapeDtypeStruct((2, _B * _V, _P), jnp.float32),
        mesh=mesh,
        compiler_params=cp,
        scratch_types=[
            pltpu.VMEM((_V, _PH), jnp.float32),
            pltpu.VMEM((_KPAD, _CB), jnp.int32),
            pltpu.VMEM((_KPAD, _CB), jnp.float32),
            pltpu.VMEM((_CB, _P), jnp.float32),
        ],
    )
    return f(feats, nbr, wgt)


def _final_body(x_ref, c0_ref, c1_ref, w_ref, b_ref, o_ref):
    c0 = c0_ref[0]
    c1 = c1_ref[0]
    u = jnp.concatenate(
        [x_ref[...], c0[:, :_PH], c1[:, :_PH], c0[:, _PH:], c1[:, _PH:]],
        axis=-1)
    o_ref[...] = jnp.tanh(
        jnp.dot(u, w_ref[...], preferred_element_type=jnp.float32) + b_ref[...]
    )


def _final_stage(x, agg, W_out, b_out):
    xf = x.reshape(_B * _V, _F)
    out = pl.pallas_call(
        _final_body,
        out_shape=jax.ShapeDtypeStruct((_B * _V, _NF), jnp.float32),
        grid=(_B,),
        compiler_params=pltpu.CompilerParams(
            dimension_semantics=("parallel",)),
        in_specs=[
            pl.BlockSpec((_V, _F), lambda i: (i, 0)),
            pl.BlockSpec((1, _V, _P), lambda i: (0, i, 0)),
            pl.BlockSpec((1, _V, _P), lambda i: (1, i, 0)),
            pl.BlockSpec((_F + 2 * _P, _NF), lambda i: (0, 0)),
            pl.BlockSpec((1, _NF), lambda i: (0, 0)),
        ],
        out_specs=pl.BlockSpec((_V, _NF), lambda i: (i, 0)),
    )(xf, agg, agg, W_out, b_out.reshape(1, _NF))
    return out.reshape(_B, _V, _NF)


def kernel(x, W_flr, b_flr, W_s, b_s, W_out, b_out):
    feats, nbr, wgt = _topk_stage(x, W_s, b_s, W_flr, b_flr)
    agg = _agg_stage(feats, nbr, wgt)
    return _final_stage(x, agg, W_out, b_out)


# speedup vs baseline: 15.7445x; 1.7548x over previous
import dataclasses
import functools

import jax
import jax.numpy as jnp
from jax.experimental import pallas as pl
from jax.experimental.pallas import tpu as pltpu
from jax.experimental.pallas import tpu_sc as plsc

_B, _V, _F = 8, 2048, 64
_K = 40
_KN = _K - 1
_D = 4
_NF = 64
_P = 32
_R = 512
_T = _V // _R
_KPAD = 64

_IDX_BITS = 11
_IDX_MASK = (1 << _IDX_BITS) - 1
_KEY_MAX = 0x7FFFFFFF


def _topk_body(x_ref, xt_ref, ws_ref, bs_ref, wf_ref, bf_ref,
               feats_ref, nbr_ref, wgt_ref):
    t = pl.program_id(1)
    x_all = x_ref[0]

    f = jnp.dot(xt_ref[0], wf_ref[...],
                preferred_element_type=jnp.float32) + bf_ref[...]
    feats_ref[0, 0] = f[:, :_P // 2]
    feats_ref[0, 1] = f[:, _P // 2:]

    coords = jnp.dot(x_all, ws_ref[...],
                     preferred_element_type=jnp.float32) + bs_ref[...]
    ct = jnp.dot(xt_ref[0], ws_ref[...],
                 preferred_element_type=jnp.float32) + bs_ref[...]
    ctT = ct.T
    daT = jnp.sum(ctT * ctT, axis=0, keepdims=True)
    db = jnp.sum(coords * coords, axis=1, keepdims=True)
    def _round_bf16(v):
        b = jax.lax.bitcast_convert_type(v, jnp.int32)
        r = (b + 0x7FFF + jnp.bitwise_and(jax.lax.shift_right_logical(b, 16), 1))
        r = jnp.bitwise_and(r, jnp.int32(-65536))
        return jax.lax.bitcast_convert_type(r, jnp.float32)

    cb = _round_bf16(coords)
    ctbT = _round_bf16(ctT)
    cross = cb[:, 0:1] * ctbT[0:1, :]
    for d in range(1, _D):
        cross = cross + cb[:, d:d + 1] * ctbT[d:d + 1, :]
    dist = jnp.maximum(daT + db - 2.0 * cross, 0.0)

    cols = jax.lax.broadcasted_iota(jnp.int32, (_V, _R), 0)
    bits = jax.lax.bitcast_convert_type(dist, jnp.int32)
    keys = jnp.bitwise_or(jnp.bitwise_and(bits, ~jnp.int32(_IDX_MASK)), cols)

    m_prev = jnp.full((1, _R), -1, jnp.int32)
    for k in range(_K):
        cand = jnp.where(keys > m_prev, keys, _KEY_MAX)
        m = jnp.min(cand, axis=0, keepdims=True)
        if k > 0:
            idx = jnp.bitwise_and(m, jnp.int32(_IDX_MASK))
            dval = jax.lax.bitcast_convert_type(
                jnp.bitwise_and(m, ~jnp.int32(_IDX_MASK)), jnp.float32)
            w = jnp.exp(-10.0 * dval)
            nbr_ref[0, k - 1, :] = idx[0]
            wgt_ref[0, k - 1, :] = w[0]
        m_prev = m


def _topk_stage(x, W_s, b_s, W_flr, b_flr):
    return pl.pallas_call(
        _topk_body,
        grid=(_B, _T),
        compiler_params=pltpu.CompilerParams(
            dimension_semantics=("parallel", "parallel")),
        in_specs=[
            pl.BlockSpec((1, _V, _F), lambda b, t: (b, 0, 0)),
            pl.BlockSpec((1, _R, _F), lambda b, t: (b, t, 0)),
            pl.BlockSpec((_F, _D), lambda b, t: (0, 0)),
            pl.BlockSpec((1, _D), lambda b, t: (0, 0)),
            pl.BlockSpec((_F, _P), lambda b, t: (0, 0)),
            pl.BlockSpec((1, _P), lambda b, t: (0, 0)),
        ],
        out_specs=[
            pl.BlockSpec((1, 2, _R, _P // 2), lambda b, t: (b, 0, t, 0)),
            pl.BlockSpec((1, _KPAD, _R), lambda b, t: (b, 0, t)),
            pl.BlockSpec((1, _KPAD, _R), lambda b, t: (b, 0, t)),
        ],
        out_shape=[
            jax.ShapeDtypeStruct((_B, 2, _V, _P // 2), jnp.float32),
            jax.ShapeDtypeStruct((_B, _KPAD, _V), jnp.int32),
            jax.ShapeDtypeStruct((_B, _KPAD, _V), jnp.float32),
        ],
    )(x, x, W_s, b_s.reshape(1, _D), W_flr, b_flr.reshape(1, _P))


_NC = 2
_NS = 16
_NW = _NC * _NS
_PH = _P // 2
_SEG = 1024
_G = 16


_CB = 128


_PB = 8


def _agg_sc_body(feats_hbm, nbr_hbm, wgt_hbm, out_hbm,
                 table_v, idx_v, wgt_v, out_v):
    c = jax.lax.axis_index("c")
    s = jax.lax.axis_index("s")
    wid = s * _NC + c
    b = wid // 4
    half = (wid // 2) % 2
    vbase = (wid % 2) * _SEG
    pltpu.sync_copy(feats_hbm.at[b, half], table_v)
    lane = jax.lax.broadcasted_iota(jnp.int32, (_G,), 0)

    @pl.loop(0, _SEG // _CB)
    def _(cchunk):
        cb = vbase + cchunk * _CB
        pltpu.sync_copy(nbr_hbm.at[b, pl.ds(0, _KPAD), pl.ds(cb, _CB)], idx_v)
        pltpu.sync_copy(wgt_hbm.at[b, pl.ds(0, _KPAD), pl.ds(cb, _CB)], wgt_v)

        @pl.loop(0, _CB // _G)
        def _(sub):
            col = sub * _G
            rows = lane + col
            for pb in range(_PH // _PB):
                def kbody(k, accs):
                    idx16 = idx_v[k, pl.ds(col, _G)]
                    w16 = wgt_v[k, pl.ds(col, _G)]
                    new = []
                    for j in range(_PB):
                        p = pb * _PB + j
                        g = plsc.load_gather(
                            table_v, [idx16, jnp.full((_G,), p, jnp.int32)])
                        wf = w16 * g
                        new.append(jnp.maximum(accs[2 * j], wf))
                        new.append(accs[2 * j + 1] + wf)
                    return tuple(new)

                init = []
                for j in range(_PB):
                    init.append(jnp.full((_G,), -jnp.inf, jnp.float32))
                    init.append(jnp.zeros((_G,), jnp.float32))
                accs = jax.lax.fori_loop(0, _KN, kbody, tuple(init))
                for j in range(_PB):
                    p = pb * _PB + j
                    plsc.store_scatter(
                        out_v, [rows, jnp.full((_G,), p, jnp.int32)],
                        accs[2 * j])
                    plsc.store_scatter(
                        out_v, [rows, jnp.full((_G,), _PH + p, jnp.int32)],
                        accs[2 * j + 1] * (1.0 / _KN))

        pltpu.sync_copy(out_v, out_hbm.at[half, pl.ds(b * _V + cb, _CB)])


def _agg_stage(feats, nbr, wgt):
    mesh = plsc.VectorSubcoreMesh(core_axis_name="c", subcore_axis_name="s")
    cp = pltpu.CompilerParams()
    if "needs_layout_passes" in pltpu.CompilerParams.__dataclass_fields__:
        cp = dataclasses.replace(cp, needs_layout_passes=False)
    if "use_tc_tiling_on_sc" in pltpu.CompilerParams.__dataclass_fields__:
        cp = dataclasses.replace(cp, use_tc_tiling_on_sc=False)
    f = pl.kernel(
        _agg_sc_body,
        out_type=jax.ShapeDtypeStruct((2, _B * _V, _P), jnp.float32),
        mesh=mesh,
        compiler_params=cp,
        scratch_types=[
            pltpu.VMEM((_V, _PH), jnp.float32),
            pltpu.VMEM((_KPAD, _CB), jnp.int32),
            pltpu.VMEM((_KPAD, _CB), jnp.float32),
            pltpu.VMEM((_CB, _P), jnp.float32),
        ],
    )
    return f(feats, nbr, wgt)


def _final_body(x_ref, c0_ref, c1_ref, w_ref, b_ref, o_ref):
    c0 = c0_ref[0]
    c1 = c1_ref[0]
    u = jnp.concatenate(
        [x_ref[...], c0[:, :_PH], c1[:, :_PH], c0[:, _PH:], c1[:, _PH:]],
        axis=-1)
    o_ref[...] = jnp.tanh(
        jnp.dot(u, w_ref[...], preferred_element_type=jnp.float32) + b_ref[...]
    )


def _final_stage(x, agg, W_out, b_out):
    xf = x.reshape(_B * _V, _F)
    out = pl.pallas_call(
        _final_body,
        out_shape=jax.ShapeDtypeStruct((_B * _V, _NF), jnp.float32),
        grid=(_B,),
        compiler_params=pltpu.CompilerParams(
            dimension_semantics=("parallel",)),
        in_specs=[
            pl.BlockSpec((_V, _F), lambda i: (i, 0)),
            pl.BlockSpec((1, _V, _P), lambda i: (0, i, 0)),
            pl.BlockSpec((1, _V, _P), lambda i: (1, i, 0)),
            pl.BlockSpec((_F + 2 * _P, _NF), lambda i: (0, 0)),
            pl.BlockSpec((1, _NF), lambda i: (0, 0)),
        ],
        out_specs=pl.BlockSpec((_V, _NF), lambda i: (i, 0)),
    )(xf, agg, agg, W_out, b_out.reshape(1, _NF))
    return out.reshape(_B, _V, _NF)


def kernel(x, W_flr, b_flr, W_s, b_s, W_out, b_out):
    feats, nbr, wgt = _topk_stage(x, W_s, b_s, W_flr, b_flr)
    agg = _agg_stage(feats, nbr, wgt)
    return _final_stage(x, agg, W_out, b_out)
